# Initial kernel scaffold; baseline (speedup 1.0000x reference)
import jax
import jax.numpy as jnp
from jax.experimental import pallas as pl


def kernel(x, edge_index, edge_weight, W_gc1, b_gc1, W_sl1, b_sl1, bn1_gamma, bn1_beta, W_gc2, b_gc2, W_sl2, b_sl2, bn2_gamma, bn2_beta, W_gc3, b_gc3, W_sl3, b_sl3):
    raise NotImplementedError("write your pallas kernel here")


# speedup vs baseline: 6.1769x; 6.1769x over previous
import functools

import jax
import jax.numpy as jnp
from jax import lax
from jax.experimental import pallas as pl
from jax.experimental.pallas import tpu as pltpu
from jax.experimental.pallas import tpu_sc as plsc

N = 10000
N_PAD = 10240
F = 128
E = 320000
NSUB = 16
CHUNK = 128
SUPER = 16
NSUP = 10
CPS = SUPER * NSUP
E_PAD = NSUB * CPS * CHUNK
ROWS_PER_SUB = N_PAD // NSUB


def _spmm_sc(support, src, dst, w):
    mesh = plsc.VectorSubcoreMesh(core_axis_name="c", subcore_axis_name="s")

    @functools.partial(
        pl.kernel,
        out_type=jax.ShapeDtypeStruct((2, N_PAD, F), jnp.float32),
        mesh=mesh,
        scratch_types=[
            pltpu.VMEM_SHARED((N_PAD, F), jnp.float32),
            pltpu.VMEM((CHUNK, F), jnp.float32),
            pltpu.VMEM((SUPER, CHUNK), jnp.int32),
            pltpu.VMEM((SUPER, CHUNK), jnp.int32),
            pltpu.VMEM((SUPER, CHUNK), jnp.float32),
        ],
    )
    def k(sup_hbm, src_hbm, dst_hbm, w_hbm, out_hbm, acc, rows, sidx, didx, wall):
        c = lax.axis_index("c")
        s = lax.axis_index("s")

        zero = jnp.zeros((1, 16), jnp.float32)

        @pl.loop(0, CHUNK)
        def _(i):
            for g in range(8):
                rows.at[pl.ds(i, 1), pl.ds(g * 16, 16)][...] = zero

        for t in range(ROWS_PER_SUB // CHUNK):
            pltpu.sync_copy(rows, acc.at[pl.ds(s * ROWS_PER_SUB + t * CHUNK, CHUNK)])
        plsc.subcore_barrier()

        sup_c = sup_hbm.at[c]
        src_s = src_hbm.at[s]
        dst_s = dst_hbm.at[s]
        w_s = w_hbm.at[s]

        @pl.loop(0, NSUP)
        def _(u):
            blk = pl.ds(u * SUPER, SUPER)
            pltpu.sync_copy(src_s.at[blk], sidx)
            pltpu.sync_copy(dst_s.at[blk], didx)
            pltpu.sync_copy(w_s.at[blk], wall)

            @pl.loop(0, SUPER)
            def _(j):
                pltpu.sync_copy(sup_c.at[sidx.at[j]], rows)

                @pl.loop(0, CHUNK, step=16)
                def _(i0):
                    wvec = wall[j, pl.ds(i0, 16)]
                    for kk in range(16):
                        wv = wvec[kk]
                        for g in range(8):
                            slc = (pl.ds(i0 + kk, 1), pl.ds(g * 16, 16))
                            rows.at[slc][...] = rows.at[slc][...] * wv

                pltpu.sync_copy(rows, acc.at[didx.at[j]], add=True)

        plsc.subcore_barrier()

        out_c = out_hbm.at[c]
        for t in range(ROWS_PER_SUB // CHUNK):
            sl = pl.ds(s * ROWS_PER_SUB + t * CHUNK, CHUNK)
            pltpu.sync_copy(acc.at[sl], out_c.at[sl])

    return k(support, src, dst, w)


def _mm2(x_ref, wg_ref, ws_ref, bias_ref, sup_ref, sl_ref):
    xf = x_ref[...].reshape(2 * N, F)
    sup = jnp.dot(xf, wg_ref[...], preferred_element_type=jnp.float32)
    sup_ref[...] = _pad_batch(sup)
    sl = jnp.dot(xf, ws_ref[...], preferred_element_type=jnp.float32) + bias_ref[...]
    sl_ref[...] = sl.reshape(2, N, F)


def _pad_batch(y2d):
    y = y2d.reshape(2, N, F)
    return jnp.pad(y, ((0, 0), (0, N_PAD - N), (0, 0)))


def _stats(gc_ref, sl_ref, y_ref, mean_ref, rstd_ref):
    y = gc_ref[...][:, :N, :] + sl_ref[...]
    mean = jnp.mean(y, axis=(0, 1))
    var = jnp.mean(y * y, axis=(0, 1)) - mean * mean
    y_ref[...] = y
    mean_ref[...] = mean.reshape(1, F)
    rstd_ref[...] = (1.0 / jnp.sqrt(var + 1e-5)).reshape(1, F)


def _apply_mm(y_ref, mean_ref, rstd_ref, gam_ref, bet_ref, hprev_ref,
              wg_ref, ws_ref, bias_ref, h_ref, sup_ref, sl_ref, *, resid, fo):
    y = y_ref[...]
    h = jax.nn.relu((y - mean_ref[...].reshape(1, 1, F))
                    * (rstd_ref[...] * gam_ref[...]).reshape(1, 1, F)
                    + bet_ref[...].reshape(1, 1, F))
    if resid:
        h = h + hprev_ref[...]
    h_ref[...] = h
    hf = h.reshape(2 * N, F)
    sup_ref[...] = _pad_batch(jnp.dot(hf, wg_ref[...],
                                      preferred_element_type=jnp.float32))
    sl = jnp.dot(hf, ws_ref[...], preferred_element_type=jnp.float32) + bias_ref[...]
    sl_ref[...] = sl.reshape(2, N, fo)


def _final(gc_ref, sl_ref, o_ref):
    o_ref[...] = jax.nn.sigmoid(gc_ref[...][:, :N, :40] + sl_ref[...])


def _tc(fn, out_shapes, *args):
    return pl.pallas_call(fn, out_shape=out_shapes)(*args)


def kernel(x, edge_index, edge_weight, W_gc1, b_gc1, W_sl1, b_sl1, bn1_gamma,
           bn1_beta, W_gc2, b_gc2, W_sl2, b_sl2, bn2_gamma, bn2_beta, W_gc3,
           b_gc3, W_sl3, b_sl3):
    f32 = jnp.float32
    pad = E_PAD - E
    src = edge_index[0].astype(jnp.int32)
    dst = edge_index[1].astype(jnp.int32)
    spread = (jnp.arange(pad, dtype=jnp.int32) * 97) % N
    src = jnp.concatenate([src, spread]).reshape(NSUB, CPS, CHUNK)
    dst = jnp.concatenate([dst, spread]).reshape(NSUB, CPS, CHUNK)
    w = jnp.concatenate([edge_weight.astype(f32),
                         jnp.zeros((pad,), f32)]).reshape(NSUB, CPS, CHUNK)

    sds = jax.ShapeDtypeStruct
    yS = sds((2, N, F), f32)
    supS = sds((2, N_PAD, F), f32)
    statS = sds((1, F), f32)

    sup1, sl1 = _tc(_mm2, [supS, yS], x, W_gc1, W_sl1,
                    (b_gc1 + b_sl1).reshape(1, F))
    gc1 = _spmm_sc(sup1, src, dst, w)
    y1, m1, r1 = _tc(_stats, [yS, statS, statS], gc1, sl1)
    h1, sup2, sl2 = _tc(
        functools.partial(_apply_mm, resid=False, fo=F),
        [yS, supS, yS],
        y1, m1, r1, bn1_gamma.reshape(1, F), bn1_beta.reshape(1, F), y1,
        W_gc2, W_sl2, (b_gc2 + b_sl2).reshape(1, F))
    gc2 = _spmm_sc(sup2, src, dst, w)
    y2, m2, r2 = _tc(_stats, [yS, statS, statS], gc2, sl2)
    W_gc3p = jnp.pad(W_gc3, ((0, 0), (0, F - 40)))
    W_sl3p = jnp.pad(W_sl3, ((0, 0), (0, F - 40)))
    b3p = jnp.pad((b_gc3 + b_sl3), (0, F - 40)).reshape(1, F)
    _, sup3, sl3p = _tc(
        functools.partial(_apply_mm, resid=True, fo=F),
        [yS, supS, yS],
        y2, m2, r2, bn2_gamma.reshape(1, F), bn2_beta.reshape(1, F), h1,
        W_gc3p, W_sl3p, b3p)
    gc3 = _spmm_sc(sup3, src, dst, w)
    out = _tc(_final, sds((2, N, 40), f32), gc3, sl3p[:, :, :40])
    return out


# speedup vs baseline: 7.8383x; 1.2690x over previous
import functools

import jax
import jax.numpy as jnp
from jax import lax
from jax.experimental import pallas as pl
from jax.experimental.pallas import tpu as pltpu
from jax.experimental.pallas import tpu_sc as plsc

N = 10000
N_PAD = 10240
F = 128
E = 320000
NSUB = 16
CHUNK = 64
SUPER = 16
NSUP = 20
CPS = SUPER * NSUP
E_PAD = NSUB * CPS * CHUNK
ROWS_PER_SUB = N_PAD // NSUB


def _spmm_sc(support, src, dst, w):
    mesh = plsc.VectorSubcoreMesh(core_axis_name="c", subcore_axis_name="s")

    @functools.partial(
        pl.kernel,
        out_type=jax.ShapeDtypeStruct((2, N_PAD, F), jnp.float32),
        mesh=mesh,
        scratch_types=[
            pltpu.VMEM_SHARED((N_PAD, F), jnp.float32),
            pltpu.VMEM((CHUNK, F), jnp.float32),
            pltpu.VMEM((CHUNK, F), jnp.float32),
            pltpu.VMEM((CHUNK, F), jnp.float32),
            pltpu.VMEM((CHUNK, F), jnp.float32),
            pltpu.VMEM((SUPER, CHUNK), jnp.int32),
            pltpu.VMEM((SUPER, CHUNK), jnp.int32),
            pltpu.VMEM((SUPER, CHUNK), jnp.float32),
            pltpu.SemaphoreType.DMA,
            pltpu.SemaphoreType.DMA,
            pltpu.SemaphoreType.DMA,
            pltpu.SemaphoreType.DMA,
        ],
    )
    def k(sup_hbm, src_hbm, dst_hbm, w_hbm, out_hbm, acc, ga, gb, sa, sb,
          sidx, didx, wall, sga, sgb, ssa, ssb):
        c = lax.axis_index("c")
        s = lax.axis_index("s")

        zero = jnp.zeros((1, 16), jnp.float32)

        @pl.loop(0, CHUNK)
        def _(i):
            for g in range(8):
                ga.at[pl.ds(i, 1), pl.ds(g * 16, 16)][...] = zero

        for t in range(ROWS_PER_SUB // CHUNK):
            pltpu.sync_copy(ga, acc.at[pl.ds(s * ROWS_PER_SUB + t * CHUNK, CHUNK)])
        plsc.subcore_barrier()

        sup_c = sup_hbm.at[c]
        src_s = src_hbm.at[s]
        dst_s = dst_hbm.at[s]
        w_s = w_hbm.at[s]

        def weight(j, gbuf, sbuf):
            @pl.loop(0, CHUNK, step=16)
            def _(i0):
                wvec = wall[j, pl.ds(i0, 16)]
                for kk in range(16):
                    wv = wvec[kk]
                    for g in range(8):
                        slc = (pl.ds(i0 + kk, 1), pl.ds(g * 16, 16))
                        sbuf.at[slc][...] = gbuf.at[slc][...] * wv

        def segment(j, gbuf, sbuf, sg, ss):
            pltpu.make_async_copy(sup_c.at[sidx.at[j]], gbuf, sg).wait()
            @pl.when(j >= 2)
            def _():
                pltpu.make_async_copy(sbuf, acc.at[didx.at[j]], ss).wait()
            weight(j, gbuf, sbuf)
            pltpu.async_copy(sbuf, acc.at[didx.at[j]], ss, add=True)
            @pl.when(j + 2 < SUPER)
            def _():
                pltpu.async_copy(sup_c.at[sidx.at[j + 2]], gbuf, sg)

        @pl.loop(0, NSUP)
        def _(u):
            pltpu.sync_copy(src_s.at[u], sidx)
            pltpu.sync_copy(dst_s.at[u], didx)
            pltpu.sync_copy(w_s.at[u], wall)
            pltpu.async_copy(sup_c.at[sidx.at[0]], ga, sga)
            pltpu.async_copy(sup_c.at[sidx.at[1]], gb, sgb)

            @pl.loop(0, SUPER, step=2)
            def _(j):
                segment(j, ga, sa, sga, ssa)
                segment(j + 1, gb, sb, sgb, ssb)

            pltpu.make_async_copy(sa, acc.at[didx.at[0]], ssa).wait()
            pltpu.make_async_copy(sb, acc.at[didx.at[0]], ssb).wait()

        plsc.subcore_barrier()

        out_c = out_hbm.at[c]
        for t in range(ROWS_PER_SUB // CHUNK):
            sl = pl.ds(s * ROWS_PER_SUB + t * CHUNK, CHUNK)
            pltpu.sync_copy(acc.at[sl], out_c.at[sl])

    return k(support, src, dst, w)


def _mm2(x_ref, wg_ref, ws_ref, bias_ref, sup_ref, sl_ref):
    xf = x_ref[...].reshape(2 * N, F)
    sup = jnp.dot(xf, wg_ref[...], preferred_element_type=jnp.float32)
    sup_ref[...] = _pad_batch(sup)
    sl = jnp.dot(xf, ws_ref[...], preferred_element_type=jnp.float32) + bias_ref[...]
    sl_ref[...] = sl.reshape(2, N, F)


def _pad_batch(y2d):
    y = y2d.reshape(2, N, F)
    return jnp.pad(y, ((0, 0), (0, N_PAD - N), (0, 0)))


def _stats(gc_ref, sl_ref, y_ref, mean_ref, rstd_ref):
    y = gc_ref[...][:, :N, :] + sl_ref[...]
    mean = jnp.mean(y, axis=(0, 1))
    var = jnp.mean(y * y, axis=(0, 1)) - mean * mean
    y_ref[...] = y
    mean_ref[...] = mean.reshape(1, F)
    rstd_ref[...] = (1.0 / jnp.sqrt(var + 1e-5)).reshape(1, F)


def _apply_mm(y_ref, mean_ref, rstd_ref, gam_ref, bet_ref, hprev_ref,
              wg_ref, ws_ref, bias_ref, h_ref, sup_ref, sl_ref, *, resid, fo):
    y = y_ref[...]
    h = jax.nn.relu((y - mean_ref[...].reshape(1, 1, F))
                    * (rstd_ref[...] * gam_ref[...]).reshape(1, 1, F)
                    + bet_ref[...].reshape(1, 1, F))
    if resid:
        h = h + hprev_ref[...]
    h_ref[...] = h
    hf = h.reshape(2 * N, F)
    sup_ref[...] = _pad_batch(jnp.dot(hf, wg_ref[...],
                                      preferred_element_type=jnp.float32))
    sl = jnp.dot(hf, ws_ref[...], preferred_element_type=jnp.float32) + bias_ref[...]
    sl_ref[...] = sl.reshape(2, N, fo)


def _final(gc_ref, sl_ref, o_ref):
    o_ref[...] = jax.nn.sigmoid(gc_ref[...][:, :N, :40] + sl_ref[...])


def _tc(fn, out_shapes, *args):
    return pl.pallas_call(fn, out_shape=out_shapes)(*args)


def kernel(x, edge_index, edge_weight, W_gc1, b_gc1, W_sl1, b_sl1, bn1_gamma,
           bn1_beta, W_gc2, b_gc2, W_sl2, b_sl2, bn2_gamma, bn2_beta, W_gc3,
           b_gc3, W_sl3, b_sl3):
    f32 = jnp.float32
    pad = E_PAD - E
    src = edge_index[0].astype(jnp.int32)
    dst = edge_index[1].astype(jnp.int32)
    eshape = (NSUB, NSUP, SUPER, CHUNK)
    spread = (jnp.arange(pad, dtype=jnp.int32) * 97) % N
    src = jnp.concatenate([src, spread]).reshape(eshape)
    dst = jnp.concatenate([dst, spread]).reshape(eshape)
    w = jnp.concatenate([edge_weight.astype(f32),
                         jnp.zeros((pad,), f32)]).reshape(eshape)

    sds = jax.ShapeDtypeStruct
    yS = sds((2, N, F), f32)
    supS = sds((2, N_PAD, F), f32)
    statS = sds((1, F), f32)

    sup1, sl1 = _tc(_mm2, [supS, yS], x, W_gc1, W_sl1,
                    (b_gc1 + b_sl1).reshape(1, F))
    gc1 = _spmm_sc(sup1, src, dst, w)
    y1, m1, r1 = _tc(_stats, [yS, statS, statS], gc1, sl1)
    h1, sup2, sl2 = _tc(
        functools.partial(_apply_mm, resid=False, fo=F),
        [yS, supS, yS],
        y1, m1, r1, bn1_gamma.reshape(1, F), bn1_beta.reshape(1, F), y1,
        W_gc2, W_sl2, (b_gc2 + b_sl2).reshape(1, F))
    gc2 = _spmm_sc(sup2, src, dst, w)
    y2, m2, r2 = _tc(_stats, [yS, statS, statS], gc2, sl2)
    W_gc3p = jnp.pad(W_gc3, ((0, 0), (0, F - 40)))
    W_sl3p = jnp.pad(W_sl3, ((0, 0), (0, F - 40)))
    b3p = jnp.pad((b_gc3 + b_sl3), (0, F - 40)).reshape(1, F)
    _, sup3, sl3p = _tc(
        functools.partial(_apply_mm, resid=True, fo=F),
        [yS, supS, yS],
        y2, m2, r2, bn2_gamma.reshape(1, F), bn2_beta.reshape(1, F), h1,
        W_gc3p, W_sl3p, b3p)
    gc3 = _spmm_sc(sup3, src, dst, w)
    out = _tc(_final, sds((2, N, 40), f32), gc3, sl3p[:, :, :40])
    return out


# speedup vs baseline: 10.0522x; 1.2825x over previous
import functools

import jax
import jax.numpy as jnp
from jax import lax
from jax.experimental import pallas as pl
from jax.experimental.pallas import tpu as pltpu
from jax.experimental.pallas import tpu_sc as plsc

N = 10000
N_PAD = 10240
F = 128
E = 320000
NSUB = 16
CHUNK = 96
SUPER = 18
NSUP = 12
CPS = SUPER * NSUP
E_PAD = NSUB * CPS * CHUNK
ROWS_PER_SUB = N_PAD // NSUB


def _spmm_sc(support, src, dst, w):
    mesh = plsc.VectorSubcoreMesh(core_axis_name="c", subcore_axis_name="s")

    @functools.partial(
        pl.kernel,
        out_type=jax.ShapeDtypeStruct((2, N_PAD, F), jnp.float32),
        mesh=mesh,
        scratch_types=[
            pltpu.VMEM_SHARED((N_PAD, F), jnp.float32),
            pltpu.VMEM((CHUNK, F), jnp.float32),
            pltpu.VMEM((CHUNK, F), jnp.float32),
            pltpu.VMEM((CHUNK, F), jnp.float32),
            pltpu.VMEM((SUPER, CHUNK), jnp.int32),
            pltpu.VMEM((SUPER, CHUNK), jnp.int32),
            pltpu.VMEM((SUPER, CHUNK), jnp.float32),
            pltpu.SemaphoreType.DMA,
            pltpu.SemaphoreType.DMA,
            pltpu.SemaphoreType.DMA,
            pltpu.SemaphoreType.DMA,
            pltpu.SemaphoreType.DMA,
            pltpu.SemaphoreType.DMA,
        ],
    )
    def k(sup_hbm, src_hbm, dst_hbm, w_hbm, out_hbm, acc, b0, b1, b2,
          sidx, didx, wall, sg0, sg1, sg2, ss0, ss1, ss2):
        c = lax.axis_index("c")
        s = lax.axis_index("s")
        bufs = (b0, b1, b2)
        sg = (sg0, sg1, sg2)
        ss = (ss0, ss1, ss2)

        zero = jnp.zeros((1, 16), jnp.float32)

        @pl.loop(0, CHUNK)
        def _(i):
            for g in range(8):
                b0.at[pl.ds(i, 1), pl.ds(g * 16, 16)][...] = zero

        base = s * ROWS_PER_SUB
        for t in range(6):
            pltpu.sync_copy(b0, acc.at[pl.ds(base + t * CHUNK, CHUNK)])
        pltpu.sync_copy(b0.at[pl.ds(0, 64)], acc.at[pl.ds(base + 6 * CHUNK, 64)])
        plsc.subcore_barrier()

        sup_c = sup_hbm.at[c]
        src_s = src_hbm.at[s]
        dst_s = dst_hbm.at[s]
        w_s = w_hbm.at[s]

        def weight(j, buf):
            @plsc.parallel_loop(0, CHUNK, step=16, unroll=2)
            def _(i0):
                wvec = wall[j, pl.ds(i0, 16)]
                for kk in range(16):
                    wv = wvec[kk]
                    for g in range(8):
                        slc = (pl.ds(i0 + kk, 1), pl.ds(g * 16, 16))
                        buf.at[slc][...] = buf.at[slc][...] * wv

        def segment(j, cur, nxt):
            @pl.when(j >= 2)
            def _():
                pltpu.make_async_copy(bufs[nxt], acc.at[didx.at[j]],
                                      ss[nxt]).wait()
            @pl.when(j + 1 < SUPER)
            def _():
                pltpu.async_copy(sup_c.at[sidx.at[j + 1]], bufs[nxt], sg[nxt])
            pltpu.make_async_copy(sup_c.at[sidx.at[j]], bufs[cur],
                                  sg[cur]).wait()
            weight(j, bufs[cur])
            pltpu.async_copy(bufs[cur], acc.at[didx.at[j]], ss[cur], add=True)

        @pl.loop(0, NSUP)
        def _(u):
            pltpu.sync_copy(src_s.at[u], sidx)
            pltpu.sync_copy(dst_s.at[u], didx)
            pltpu.sync_copy(w_s.at[u], wall)
            pltpu.async_copy(sup_c.at[sidx.at[0]], b0, sg0)

            @pl.loop(0, SUPER, step=3)
            def _(j):
                segment(j, 0, 1)
                segment(j + 1, 1, 2)
                segment(j + 2, 2, 0)

            pltpu.make_async_copy(b1, acc.at[didx.at[0]], ss1).wait()
            pltpu.make_async_copy(b2, acc.at[didx.at[0]], ss2).wait()

        plsc.subcore_barrier()

        out_c = out_hbm.at[c]
        for t in range(6):
            sl = pl.ds(base + t * CHUNK, CHUNK)
            pltpu.sync_copy(acc.at[sl], out_c.at[sl])
        sl = pl.ds(base + 6 * CHUNK, 64)
        pltpu.sync_copy(acc.at[sl], out_c.at[sl])

    return k(support, src, dst, w)


def _mm2(x_ref, wg_ref, ws_ref, bias_ref, sup_ref, sl_ref):
    xf = x_ref[...].reshape(2 * N, F)
    sup = jnp.dot(xf, wg_ref[...], preferred_element_type=jnp.float32)
    sup_ref[...] = _pad_batch(sup)
    sl = jnp.dot(xf, ws_ref[...], preferred_element_type=jnp.float32) + bias_ref[...]
    sl_ref[...] = sl.reshape(2, N, F)


def _pad_batch(y2d):
    y = y2d.reshape(2, N, F)
    return jnp.pad(y, ((0, 0), (0, N_PAD - N), (0, 0)))


def _stats(gc_ref, sl_ref, y_ref, mean_ref, rstd_ref):
    y = gc_ref[...][:, :N, :] + sl_ref[...]
    mean = jnp.mean(y, axis=(0, 1))
    var = jnp.mean(y * y, axis=(0, 1)) - mean * mean
    y_ref[...] = y
    mean_ref[...] = mean.reshape(1, F)
    rstd_ref[...] = (1.0 / jnp.sqrt(var + 1e-5)).reshape(1, F)


def _apply_mm(y_ref, mean_ref, rstd_ref, gam_ref, bet_ref, hprev_ref,
              wg_ref, ws_ref, bias_ref, h_ref, sup_ref, sl_ref, *, resid, fo):
    y = y_ref[...]
    h = jax.nn.relu((y - mean_ref[...].reshape(1, 1, F))
                    * (rstd_ref[...] * gam_ref[...]).reshape(1, 1, F)
                    + bet_ref[...].reshape(1, 1, F))
    if resid:
        h = h + hprev_ref[...]
    h_ref[...] = h
    hf = h.reshape(2 * N, F)
    sup_ref[...] = _pad_batch(jnp.dot(hf, wg_ref[...],
                                      preferred_element_type=jnp.float32))
    sl = jnp.dot(hf, ws_ref[...], preferred_element_type=jnp.float32) + bias_ref[...]
    sl_ref[...] = sl.reshape(2, N, fo)


def _final(gc_ref, sl_ref, o_ref):
    o_ref[...] = jax.nn.sigmoid(gc_ref[...][:, :N, :40] + sl_ref[...])


def _tc(fn, out_shapes, *args):
    return pl.pallas_call(fn, out_shape=out_shapes)(*args)


def kernel(x, edge_index, edge_weight, W_gc1, b_gc1, W_sl1, b_sl1, bn1_gamma,
           bn1_beta, W_gc2, b_gc2, W_sl2, b_sl2, bn2_gamma, bn2_beta, W_gc3,
           b_gc3, W_sl3, b_sl3):
    f32 = jnp.float32
    pad = E_PAD - E
    src = edge_index[0].astype(jnp.int32)
    dst = edge_index[1].astype(jnp.int32)
    eshape = (NSUB, NSUP, SUPER, CHUNK)
    spread = (jnp.arange(pad, dtype=jnp.int32) * 97) % N
    src = jnp.concatenate([src, spread]).reshape(eshape)
    dst = jnp.concatenate([dst, spread]).reshape(eshape)
    w = jnp.concatenate([edge_weight.astype(f32),
                         jnp.zeros((pad,), f32)]).reshape(eshape)

    sds = jax.ShapeDtypeStruct
    yS = sds((2, N, F), f32)
    supS = sds((2, N_PAD, F), f32)
    statS = sds((1, F), f32)

    sup1, sl1 = _tc(_mm2, [supS, yS], x, W_gc1, W_sl1,
                    (b_gc1 + b_sl1).reshape(1, F))
    gc1 = _spmm_sc(sup1, src, dst, w)
    y1, m1, r1 = _tc(_stats, [yS, statS, statS], gc1, sl1)
    h1, sup2, sl2 = _tc(
        functools.partial(_apply_mm, resid=False, fo=F),
        [yS, supS, yS],
        y1, m1, r1, bn1_gamma.reshape(1, F), bn1_beta.reshape(1, F), y1,
        W_gc2, W_sl2, (b_gc2 + b_sl2).reshape(1, F))
    gc2 = _spmm_sc(sup2, src, dst, w)
    y2, m2, r2 = _tc(_stats, [yS, statS, statS], gc2, sl2)
    W_gc3p = jnp.pad(W_gc3, ((0, 0), (0, F - 40)))
    W_sl3p = jnp.pad(W_sl3, ((0, 0), (0, F - 40)))
    b3p = jnp.pad((b_gc3 + b_sl3), (0, F - 40)).reshape(1, F)
    _, sup3, sl3p = _tc(
        functools.partial(_apply_mm, resid=True, fo=F),
        [yS, supS, yS],
        y2, m2, r2, bn2_gamma.reshape(1, F), bn2_beta.reshape(1, F), h1,
        W_gc3p, W_sl3p, b3p)
    gc3 = _spmm_sc(sup3, src, dst, w)
    out = _tc(_final, sds((2, N, 40), f32), gc3, sl3p[:, :, :40])
    return out


# speedup vs baseline: 10.1226x; 1.0070x over previous
import functools

import jax
import jax.numpy as jnp
from jax import lax
from jax.experimental import pallas as pl
from jax.experimental.pallas import tpu as pltpu
from jax.experimental.pallas import tpu_sc as plsc

N = 10000
N_PAD = 10240
F = 128
E = 320000
NSUB = 16
CHUNK = 96
SUPER = 9
NSUP = 24
CPS = SUPER * NSUP
E_PAD = NSUB * CPS * CHUNK
ROWS_PER_SUB = N_PAD // NSUB


def _spmm_sc(support, src, dst, w):
    mesh = plsc.VectorSubcoreMesh(core_axis_name="c", subcore_axis_name="s")

    @functools.partial(
        pl.kernel,
        out_type=jax.ShapeDtypeStruct((2, N_PAD, F), jnp.float32),
        mesh=mesh,
        scratch_types=[
            pltpu.VMEM_SHARED((N_PAD, F), jnp.float32),
            pltpu.VMEM((CHUNK, F), jnp.float32),
            pltpu.VMEM((CHUNK, F), jnp.float32),
            pltpu.VMEM((CHUNK, F), jnp.float32),
            pltpu.VMEM((SUPER, CHUNK), jnp.int32),
            pltpu.VMEM((SUPER, CHUNK), jnp.int32),
            pltpu.VMEM((SUPER, CHUNK), jnp.float32),
            pltpu.VMEM((SUPER, CHUNK), jnp.int32),
            pltpu.VMEM((SUPER, CHUNK), jnp.int32),
            pltpu.VMEM((SUPER, CHUNK), jnp.float32),
            pltpu.SemaphoreType.DMA,
            pltpu.SemaphoreType.DMA,
            pltpu.SemaphoreType.DMA,
            pltpu.SemaphoreType.DMA,
            pltpu.SemaphoreType.DMA,
            pltpu.SemaphoreType.DMA,
            pltpu.SemaphoreType.DMA,
            pltpu.SemaphoreType.DMA,
        ],
    )
    def k(sup_hbm, src_hbm, dst_hbm, w_hbm, out_hbm, acc, b0, b1, b2,
          sidxA, didxA, wallA, sidxB, didxB, wallB,
          sg0, sg1, sg2, ss0, ss1, ss2, stA, stB):
        c = lax.axis_index("c")
        s = lax.axis_index("s")
        bufs = (b0, b1, b2)
        sg = (sg0, sg1, sg2)
        ss = (ss0, ss1, ss2)

        zero = jnp.zeros((1, 16), jnp.float32)

        @pl.loop(0, CHUNK)
        def _(i):
            for g in range(8):
                b0.at[pl.ds(i, 1), pl.ds(g * 16, 16)][...] = zero

        base = s * ROWS_PER_SUB
        for t in range(6):
            pltpu.sync_copy(b0, acc.at[pl.ds(base + t * CHUNK, CHUNK)])
        pltpu.sync_copy(b0.at[pl.ds(0, 64)], acc.at[pl.ds(base + 6 * CHUNK, 64)])
        plsc.subcore_barrier()

        sup_c = sup_hbm.at[c]
        src_s = src_hbm.at[s]
        dst_s = dst_hbm.at[s]
        w_s = w_hbm.at[s]

        def stage(u, sidx, didx, wall, sem):
            pltpu.async_copy(src_s.at[u], sidx, sem)
            pltpu.async_copy(dst_s.at[u], didx, sem)
            pltpu.async_copy(w_s.at[u], wall, sem)

        def stage_wait(u, sidx, didx, wall, sem):
            pltpu.make_async_copy(src_s.at[u], sidx, sem).wait()
            pltpu.make_async_copy(dst_s.at[u], didx, sem).wait()
            pltpu.make_async_copy(w_s.at[u], wall, sem).wait()

        def weight(j, wall, buf):
            @plsc.parallel_loop(0, CHUNK, step=16, unroll=2)
            def _(i0):
                wvec = wall[j, pl.ds(i0, 16)]
                for kk in range(16):
                    wv = wvec[kk]
                    for g in range(8):
                        slc = (pl.ds(i0 + kk, 1), pl.ds(g * 16, 16))
                        buf.at[slc][...] = buf.at[slc][...] * wv

        def segment(j, cur, nxt, sidx, didx, wall):
            @pl.when(j >= 2)
            def _():
                pltpu.make_async_copy(bufs[nxt], acc.at[didx.at[j]],
                                      ss[nxt]).wait()
            @pl.when(j + 1 < SUPER)
            def _():
                pltpu.async_copy(sup_c.at[sidx.at[j + 1]], bufs[nxt], sg[nxt])
            pltpu.make_async_copy(sup_c.at[sidx.at[j]], bufs[cur],
                                  sg[cur]).wait()
            weight(j, wall, bufs[cur])
            pltpu.async_copy(bufs[cur], acc.at[didx.at[j]], ss[cur], add=True)

        def superblock(u, sidx, didx, wall, sem):
            stage_wait(u, sidx, didx, wall, sem)
            pltpu.async_copy(sup_c.at[sidx.at[0]], b0, sg0)

            @pl.loop(0, SUPER, step=3)
            def _(j):
                segment(j, 0, 1, sidx, didx, wall)
                segment(j + 1, 1, 2, sidx, didx, wall)
                segment(j + 2, 2, 0, sidx, didx, wall)

            pltpu.make_async_copy(b1, acc.at[didx.at[0]], ss1).wait()
            pltpu.make_async_copy(b2, acc.at[didx.at[0]], ss2).wait()

        stage(0, sidxA, didxA, wallA, stA)
        stage(1, sidxB, didxB, wallB, stB)

        @pl.loop(0, NSUP // 2)
        def _(up):
            u = up * 2
            superblock(u, sidxA, didxA, wallA, stA)

            @pl.when(u + 2 < NSUP)
            def _():
                stage(u + 2, sidxA, didxA, wallA, stA)

            superblock(u + 1, sidxB, didxB, wallB, stB)

            @pl.when(u + 3 < NSUP)
            def _():
                stage(u + 3, sidxB, didxB, wallB, stB)

        plsc.subcore_barrier()

        out_c = out_hbm.at[c]
        for t in range(6):
            sl = pl.ds(base + t * CHUNK, CHUNK)
            pltpu.sync_copy(acc.at[sl], out_c.at[sl])
        sl = pl.ds(base + 6 * CHUNK, 64)
        pltpu.sync_copy(acc.at[sl], out_c.at[sl])

    return k(support, src, dst, w)


def _mm2(x_ref, wg_ref, ws_ref, bias_ref, sup_ref, sl_ref):
    xf = x_ref[...].reshape(2 * N, F)
    sup = jnp.dot(xf, wg_ref[...], preferred_element_type=jnp.float32)
    sup_ref[...] = _pad_batch(sup)
    sl = jnp.dot(xf, ws_ref[...], preferred_element_type=jnp.float32) + bias_ref[...]
    sl_ref[...] = sl.reshape(2, N, F)


def _pad_batch(y2d):
    y = y2d.reshape(2, N, F)
    return jnp.pad(y, ((0, 0), (0, N_PAD - N), (0, 0)))


def _stats(gc_ref, sl_ref, y_ref, mean_ref, rstd_ref):
    y = gc_ref[...][:, :N, :] + sl_ref[...]
    mean = jnp.mean(y, axis=(0, 1))
    var = jnp.mean(y * y, axis=(0, 1)) - mean * mean
    y_ref[...] = y
    mean_ref[...] = mean.reshape(1, F)
    rstd_ref[...] = (1.0 / jnp.sqrt(var + 1e-5)).reshape(1, F)


def _apply_mm(y_ref, mean_ref, rstd_ref, gam_ref, bet_ref, hprev_ref,
              wg_ref, ws_ref, bias_ref, h_ref, sup_ref, sl_ref, *, resid, fo):
    y = y_ref[...]
    h = jax.nn.relu((y - mean_ref[...].reshape(1, 1, F))
                    * (rstd_ref[...] * gam_ref[...]).reshape(1, 1, F)
                    + bet_ref[...].reshape(1, 1, F))
    if resid:
        h = h + hprev_ref[...]
    h_ref[...] = h
    hf = h.reshape(2 * N, F)
    sup_ref[...] = _pad_batch(jnp.dot(hf, wg_ref[...],
                                      preferred_element_type=jnp.float32))
    sl = jnp.dot(hf, ws_ref[...], preferred_element_type=jnp.float32) + bias_ref[...]
    sl_ref[...] = sl.reshape(2, N, fo)


def _final(gc_ref, sl_ref, o_ref):
    o_ref[...] = jax.nn.sigmoid(gc_ref[...][:, :N, :40] + sl_ref[...])


def _tc(fn, out_shapes, *args):
    return pl.pallas_call(fn, out_shape=out_shapes)(*args)


def kernel(x, edge_index, edge_weight, W_gc1, b_gc1, W_sl1, b_sl1, bn1_gamma,
           bn1_beta, W_gc2, b_gc2, W_sl2, b_sl2, bn2_gamma, bn2_beta, W_gc3,
           b_gc3, W_sl3, b_sl3):
    f32 = jnp.float32
    pad = E_PAD - E
    src = edge_index[0].astype(jnp.int32)
    dst = edge_index[1].astype(jnp.int32)
    eshape = (NSUB, NSUP, SUPER, CHUNK)
    spread = (jnp.arange(pad, dtype=jnp.int32) * 97) % N
    src = jnp.concatenate([src, spread]).reshape(eshape)
    dst = jnp.concatenate([dst, spread]).reshape(eshape)
    w = jnp.concatenate([edge_weight.astype(f32),
                         jnp.zeros((pad,), f32)]).reshape(eshape)

    sds = jax.ShapeDtypeStruct
    yS = sds((2, N, F), f32)
    supS = sds((2, N_PAD, F), f32)
    statS = sds((1, F), f32)

    sup1, sl1 = _tc(_mm2, [supS, yS], x, W_gc1, W_sl1,
                    (b_gc1 + b_sl1).reshape(1, F))
    gc1 = _spmm_sc(sup1, src, dst, w)
    y1, m1, r1 = _tc(_stats, [yS, statS, statS], gc1, sl1)
    h1, sup2, sl2 = _tc(
        functools.partial(_apply_mm, resid=False, fo=F),
        [yS, supS, yS],
        y1, m1, r1, bn1_gamma.reshape(1, F), bn1_beta.reshape(1, F), y1,
        W_gc2, W_sl2, (b_gc2 + b_sl2).reshape(1, F))
    gc2 = _spmm_sc(sup2, src, dst, w)
    y2, m2, r2 = _tc(_stats, [yS, statS, statS], gc2, sl2)
    W_gc3p = jnp.pad(W_gc3, ((0, 0), (0, F - 40)))
    W_sl3p = jnp.pad(W_sl3, ((0, 0), (0, F - 40)))
    b3p = jnp.pad((b_gc3 + b_sl3), (0, F - 40)).reshape(1, F)
    _, sup3, sl3p = _tc(
        functools.partial(_apply_mm, resid=True, fo=F),
        [yS, supS, yS],
        y2, m2, r2, bn2_gamma.reshape(1, F), bn2_beta.reshape(1, F), h1,
        W_gc3p, W_sl3p, b3p)
    gc3 = _spmm_sc(sup3, src, dst, w)
    out = _tc(_final, sds((2, N, 40), f32), gc3, sl3p[:, :, :40])
    return out


# speedup vs baseline: 10.3921x; 1.0266x over previous
import functools

import jax
import jax.numpy as jnp
from jax import lax
from jax.experimental import pallas as pl
from jax.experimental.pallas import tpu as pltpu
from jax.experimental.pallas import tpu_sc as plsc

N = 10000
N_PAD = 10240
F = 128
E = 320000
NSUB = 16
CHUNK = 96
SUPER = 12
NSUP = 18
CPS = SUPER * NSUP
E_PAD = NSUB * CPS * CHUNK
ROWS_PER_SUB = N_PAD // NSUB


def _spmm_sc(support, src, dst, w):
    mesh = plsc.VectorSubcoreMesh(core_axis_name="c", subcore_axis_name="s")

    @functools.partial(
        pl.kernel,
        out_type=jax.ShapeDtypeStruct((2, N_PAD, F), jnp.float32),
        mesh=mesh,
        scratch_types=[
            pltpu.VMEM_SHARED((N_PAD, F), jnp.float32),
            pltpu.VMEM((CHUNK, F), jnp.float32),
            pltpu.VMEM((CHUNK, F), jnp.float32),
            pltpu.VMEM((CHUNK, F), jnp.float32),
            pltpu.VMEM((SUPER, CHUNK), jnp.int32),
            pltpu.VMEM((SUPER, CHUNK), jnp.int32),
            pltpu.VMEM((SUPER, CHUNK), jnp.float32),
            pltpu.VMEM((SUPER, CHUNK), jnp.int32),
            pltpu.VMEM((SUPER, CHUNK), jnp.int32),
            pltpu.VMEM((SUPER, CHUNK), jnp.float32),
            pltpu.SemaphoreType.DMA,
            pltpu.SemaphoreType.DMA,
            pltpu.SemaphoreType.DMA,
            pltpu.SemaphoreType.DMA,
            pltpu.SemaphoreType.DMA,
            pltpu.SemaphoreType.DMA,
            pltpu.SemaphoreType.DMA,
            pltpu.SemaphoreType.DMA,
        ],
    )
    def k(sup_hbm, src_hbm, dst_hbm, w_hbm, out_hbm, acc, b0, b1, b2,
          sidxA, didxA, wallA, sidxB, didxB, wallB,
          sg0, sg1, sg2, ss0, ss1, ss2, stA, stB):
        c = lax.axis_index("c")
        s = lax.axis_index("s")
        bufs = (b0, b1, b2)
        sg = (sg0, sg1, sg2)
        ss = (ss0, ss1, ss2)

        zero = jnp.zeros((1, 16), jnp.float32)

        @pl.loop(0, CHUNK)
        def _(i):
            for g in range(8):
                b0.at[pl.ds(i, 1), pl.ds(g * 16, 16)][...] = zero

        base = s * ROWS_PER_SUB
        for t in range(6):
            pltpu.sync_copy(b0, acc.at[pl.ds(base + t * CHUNK, CHUNK)])
        pltpu.sync_copy(b0.at[pl.ds(0, 64)], acc.at[pl.ds(base + 6 * CHUNK, 64)])
        plsc.subcore_barrier()

        sup_c = sup_hbm.at[c]
        src_s = src_hbm.at[s]
        dst_s = dst_hbm.at[s]
        w_s = w_hbm.at[s]

        def stage(u, sidx, didx, wall, sem):
            pltpu.async_copy(src_s.at[u], sidx, sem)
            pltpu.async_copy(dst_s.at[u], didx, sem)
            pltpu.async_copy(w_s.at[u], wall, sem)

        def stage_wait(u, sidx, didx, wall, sem):
            pltpu.make_async_copy(src_s.at[u], sidx, sem).wait()
            pltpu.make_async_copy(dst_s.at[u], didx, sem).wait()
            pltpu.make_async_copy(w_s.at[u], wall, sem).wait()

        def weight(j, wall, buf):
            @plsc.parallel_loop(0, CHUNK, step=16, unroll=2)
            def _(i0):
                wvec = wall[j, pl.ds(i0, 16)]
                for kk in range(16):
                    wv = wvec[kk]
                    for g in range(8):
                        slc = (pl.ds(i0 + kk, 1), pl.ds(g * 16, 16))
                        buf.at[slc][...] = buf.at[slc][...] * wv

        def segment(j, cur, nxt, sidx, didx, wall):
            @pl.when(j >= 2)
            def _():
                pltpu.make_async_copy(bufs[nxt], acc.at[didx.at[j]],
                                      ss[nxt]).wait()
            @pl.when(j + 1 < SUPER)
            def _():
                pltpu.async_copy(sup_c.at[sidx.at[j + 1]], bufs[nxt], sg[nxt])
            pltpu.make_async_copy(sup_c.at[sidx.at[j]], bufs[cur],
                                  sg[cur]).wait()
            weight(j, wall, bufs[cur])
            pltpu.async_copy(bufs[cur], acc.at[didx.at[j]], ss[cur], add=True)

        def superblock(u, sidx, didx, wall, sem):
            stage_wait(u, sidx, didx, wall, sem)
            pltpu.async_copy(sup_c.at[sidx.at[0]], b0, sg0)

            @pl.loop(0, SUPER, step=3)
            def _(j):
                segment(j, 0, 1, sidx, didx, wall)
                segment(j + 1, 1, 2, sidx, didx, wall)
                segment(j + 2, 2, 0, sidx, didx, wall)

            pltpu.make_async_copy(b1, acc.at[didx.at[0]], ss1).wait()
            pltpu.make_async_copy(b2, acc.at[didx.at[0]], ss2).wait()

        stage(0, sidxA, didxA, wallA, stA)
        stage(1, sidxB, didxB, wallB, stB)

        @pl.loop(0, NSUP // 2)
        def _(up):
            u = up * 2
            superblock(u, sidxA, didxA, wallA, stA)

            @pl.when(u + 2 < NSUP)
            def _():
                stage(u + 2, sidxA, didxA, wallA, stA)

            superblock(u + 1, sidxB, didxB, wallB, stB)

            @pl.when(u + 3 < NSUP)
            def _():
                stage(u + 3, sidxB, didxB, wallB, stB)

        plsc.subcore_barrier()

        out_c = out_hbm.at[c]
        for t in range(6):
            sl = pl.ds(base + t * CHUNK, CHUNK)
            pltpu.sync_copy(acc.at[sl], out_c.at[sl])
        sl = pl.ds(base + 6 * CHUNK, 64)
        pltpu.sync_copy(acc.at[sl], out_c.at[sl])

    return k(support, src, dst, w)


def _mm2(x_ref, wg_ref, ws_ref, bias_ref, sup_ref, sl_ref):
    xf = x_ref[...].reshape(2 * N, F)
    sup = jnp.dot(xf, wg_ref[...], preferred_element_type=jnp.float32)
    sup_ref[...] = _pad_batch(sup)
    sl = jnp.dot(xf, ws_ref[...], preferred_element_type=jnp.float32) + bias_ref[...]
    sl_ref[...] = sl.reshape(2, N, F)


def _pad_batch(y2d):
    y = y2d.reshape(2, N, F)
    return jnp.pad(y, ((0, 0), (0, N_PAD - N), (0, 0)))


def _stats(gc_ref, sl_ref, y_ref, mean_ref, rstd_ref):
    y = gc_ref[...][:, :N, :] + sl_ref[...]
    mean = jnp.mean(y, axis=(0, 1))
    var = jnp.mean(y * y, axis=(0, 1)) - mean * mean
    y_ref[...] = y
    mean_ref[...] = mean.reshape(1, F)
    rstd_ref[...] = (1.0 / jnp.sqrt(var + 1e-5)).reshape(1, F)


def _apply_mm(y_ref, mean_ref, rstd_ref, gam_ref, bet_ref, hprev_ref,
              wg_ref, ws_ref, bias_ref, h_ref, sup_ref, sl_ref, *, resid, fo):
    y = y_ref[...]
    h = jax.nn.relu((y - mean_ref[...].reshape(1, 1, F))
                    * (rstd_ref[...] * gam_ref[...]).reshape(1, 1, F)
                    + bet_ref[...].reshape(1, 1, F))
    if resid:
        h = h + hprev_ref[...]
    h_ref[...] = h
    hf = h.reshape(2 * N, F)
    sup_ref[...] = _pad_batch(jnp.dot(hf, wg_ref[...],
                                      preferred_element_type=jnp.float32))
    sl = jnp.dot(hf, ws_ref[...], preferred_element_type=jnp.float32) + bias_ref[...]
    sl_ref[...] = sl.reshape(2, N, fo)


def _final(gc_ref, sl_ref, o_ref):
    o_ref[...] = jax.nn.sigmoid(gc_ref[...][:, :N, :40] + sl_ref[...])


def _tc(fn, out_shapes, *args):
    return pl.pallas_call(fn, out_shape=out_shapes)(*args)


def kernel(x, edge_index, edge_weight, W_gc1, b_gc1, W_sl1, b_sl1, bn1_gamma,
           bn1_beta, W_gc2, b_gc2, W_sl2, b_sl2, bn2_gamma, bn2_beta, W_gc3,
           b_gc3, W_sl3, b_sl3):
    f32 = jnp.float32
    pad = E_PAD - E
    src = edge_index[0].astype(jnp.int32)
    dst = edge_index[1].astype(jnp.int32)
    eshape = (NSUB, NSUP, SUPER, CHUNK)
    spread = (jnp.arange(pad, dtype=jnp.int32) * 97) % N
    src = jnp.concatenate([src, spread]).reshape(eshape)
    dst = jnp.concatenate([dst, spread]).reshape(eshape)
    w = jnp.concatenate([edge_weight.astype(f32),
                         jnp.zeros((pad,), f32)]).reshape(eshape)

    sds = jax.ShapeDtypeStruct
    yS = sds((2, N, F), f32)
    supS = sds((2, N_PAD, F), f32)
    statS = sds((1, F), f32)

    sup1, sl1 = _tc(_mm2, [supS, yS], x, W_gc1, W_sl1,
                    (b_gc1 + b_sl1).reshape(1, F))
    gc1 = _spmm_sc(sup1, src, dst, w)
    y1, m1, r1 = _tc(_stats, [yS, statS, statS], gc1, sl1)
    h1, sup2, sl2 = _tc(
        functools.partial(_apply_mm, resid=False, fo=F),
        [yS, supS, yS],
        y1, m1, r1, bn1_gamma.reshape(1, F), bn1_beta.reshape(1, F), y1,
        W_gc2, W_sl2, (b_gc2 + b_sl2).reshape(1, F))
    gc2 = _spmm_sc(sup2, src, dst, w)
    y2, m2, r2 = _tc(_stats, [yS, statS, statS], gc2, sl2)
    W_gc3p = jnp.pad(W_gc3, ((0, 0), (0, F - 40)))
    W_sl3p = jnp.pad(W_sl3, ((0, 0), (0, F - 40)))
    b3p = jnp.pad((b_gc3 + b_sl3), (0, F - 40)).reshape(1, F)
    _, sup3, sl3p = _tc(
        functools.partial(_apply_mm, resid=True, fo=F),
        [yS, supS, yS],
        y2, m2, r2, bn2_gamma.reshape(1, F), bn2_beta.reshape(1, F), h1,
        W_gc3p, W_sl3p, b3p)
    gc3 = _spmm_sc(sup3, src, dst, w)
    out = _tc(_final, sds((2, N, 40), f32), gc3, sl3p[:, :, :40])
    return out
